# baseline (speedup 1.0000x reference)
import functools

import jax
import jax.numpy as jnp
from jax import lax
from jax.experimental import pallas as pl
from jax.experimental.pallas import tpu as pltpu
from jax.experimental.pallas import tpu_sc as plsc

VOCAB = 1000000
EMB = 64
HID = 128
OUT = 2
L, B = 200, 4096

NC, NS = 2, 16
NW = NC * NS
BPW = B // NW
HALF = L // 2
LANES = 16
INV_L = 1.0 / L


def _mean_pool_body(xT_hbm, emb_hbm, m_hbm, idx_v, rows_v, m_v, sem):
    wid = lax.axis_index("s") * NC + lax.axis_index("c")
    base = wid * BPW

    pltpu.sync_copy(xT_hbm.at[pl.ds(base, BPW)], idx_v)

    def do_batch(b, carry):
        cp0 = pltpu.async_copy(emb_hbm.at[idx_v.at[b, 0]],
                               rows_v.at[pl.ds(0, HALF)], sem)
        cp1 = pltpu.async_copy(emb_hbm.at[idx_v.at[b, 1]],
                               rows_v.at[pl.ds(HALF, HALF)], sem)
        cp0.wait()
        cp1.wait()

        def red(r, accs):
            return tuple(a + rows_v[r, pl.ds(c * LANES, LANES)]
                         for c, a in enumerate(accs))

        zeros = tuple(jnp.zeros((LANES,), jnp.float32)
                      for _ in range(EMB // LANES))
        accs = lax.fori_loop(0, L, red, zeros)
        for c, a in enumerate(accs):
            m_v[b, pl.ds(c * LANES, LANES)] = a * INV_L
        return carry

    lax.fori_loop(0, BPW, do_batch, 0)

    pltpu.sync_copy(m_v, m_hbm.at[pl.ds(base, BPW)])


@functools.partial(
    pl.kernel,
    out_type=jax.ShapeDtypeStruct((B, EMB), jnp.float32),
    mesh=plsc.VectorSubcoreMesh(core_axis_name="c", subcore_axis_name="s"),
    scratch_types=[
        pltpu.VMEM((BPW, 2, HALF), jnp.int32),
        pltpu.VMEM((L, EMB), jnp.float32),
        pltpu.VMEM((BPW, EMB), jnp.float32),
        pltpu.SemaphoreType.DMA,
    ],
    compiler_params=pltpu.CompilerParams(use_tc_tiling_on_sc=False),
)
def _mean_pool(xT_hbm, emb_hbm, m_hbm, idx_v, rows_v, m_v, sem):
    _mean_pool_body(xT_hbm, emb_hbm, m_hbm, idx_v, rows_v, m_v, sem)


def _mlp_body(m_ref, w1_ref, b1_ref, w2_ref, b2_ref, o_ref):
    h = jnp.dot(m_ref[...], w1_ref[...], preferred_element_type=jnp.float32)
    h = jnp.maximum(h + b1_ref[...], 0.0)
    o_ref[...] = jnp.dot(h, w2_ref[...],
                         preferred_element_type=jnp.float32) + b2_ref[...]


_mlp = pl.pallas_call(
    _mlp_body,
    out_shape=jax.ShapeDtypeStruct((B, OUT), jnp.float32),
)


def kernel(x, emb, W1, b1, W2, b2):
    xT = x.T.reshape(B, 2, HALF)
    m = _mean_pool(xT, emb)
    return _mlp(m, W1, b1.reshape(1, HID), W2, b2.reshape(1, OUT))


# speedup vs baseline: 1.1914x; 1.1914x over previous
import functools

import jax
import jax.numpy as jnp
from jax import lax
from jax.experimental import pallas as pl
from jax.experimental.pallas import tpu as pltpu
from jax.experimental.pallas import tpu_sc as plsc

VOCAB = 1000000
EMB = 64
HID = 128
OUT = 2
L, B = 200, 4096

NC, NS = 2, 16
NW = NC * NS
BPW = B // NW
HALF = L // 2
LANES = 16
INV_L = 1.0 / L


RING = 4


def _mean_pool_body(xT_hbm, emb_hbm, m_hbm, idx_v, rows_v, m_v, *sems):
    wid = lax.axis_index("s") * NC + lax.axis_index("c")
    base = wid * BPW

    pltpu.sync_copy(xT_hbm.at[pl.ds(base, BPW)], idx_v)

    def issue(b, s):
        pltpu.async_copy(emb_hbm.at[idx_v.at[b, 0]],
                         rows_v.at[s, pl.ds(0, HALF)], sems[s])
        pltpu.async_copy(emb_hbm.at[idx_v.at[b, 1]],
                         rows_v.at[s, pl.ds(HALF, HALF)], sems[s])

    def drain(s):
        pltpu.make_async_copy(emb_hbm.at[pl.ds(0, L)],
                              rows_v.at[s], sems[s]).wait()

    for s in range(RING):
        issue(s, s)

    def outer(i, carry):
        for s in range(RING):
            b = i * RING + s
            drain(s)

            def red(r, accs):
                return tuple(a + rows_v[s, r, pl.ds(c * LANES, LANES)]
                             for c, a in enumerate(accs))

            zeros = tuple(jnp.zeros((LANES,), jnp.float32)
                          for _ in range(EMB // LANES))
            accs = lax.fori_loop(0, L, red, zeros)
            for c, a in enumerate(accs):
                m_v[b, pl.ds(c * LANES, LANES)] = a * INV_L

            nb = b + RING

            @pl.when(nb < BPW)
            def _():
                issue(nb, s)
        return carry

    lax.fori_loop(0, BPW // RING, outer, 0)

    pltpu.sync_copy(m_v, m_hbm.at[pl.ds(base, BPW)])


@functools.partial(
    pl.kernel,
    out_type=jax.ShapeDtypeStruct((B, EMB), jnp.float32),
    mesh=plsc.VectorSubcoreMesh(core_axis_name="c", subcore_axis_name="s"),
    scratch_types=[
        pltpu.VMEM((BPW, 2, HALF), jnp.int32),
        pltpu.VMEM((RING, L, EMB), jnp.float32),
        pltpu.VMEM((BPW, EMB), jnp.float32),
    ] + [pltpu.SemaphoreType.DMA] * RING,
    compiler_params=pltpu.CompilerParams(use_tc_tiling_on_sc=False),
)
def _mean_pool(xT_hbm, emb_hbm, m_hbm, idx_v, rows_v, m_v, *sems):
    _mean_pool_body(xT_hbm, emb_hbm, m_hbm, idx_v, rows_v, m_v, *sems)


def _mlp_body(m_ref, w1_ref, b1_ref, w2_ref, b2_ref, o_ref):
    h = jnp.dot(m_ref[...], w1_ref[...], preferred_element_type=jnp.float32)
    h = jnp.maximum(h + b1_ref[...], 0.0)
    o_ref[...] = jnp.dot(h, w2_ref[...],
                         preferred_element_type=jnp.float32) + b2_ref[...]


_mlp = pl.pallas_call(
    _mlp_body,
    out_shape=jax.ShapeDtypeStruct((B, OUT), jnp.float32),
)


def kernel(x, emb, W1, b1, W2, b2):
    xT = x.T.reshape(B, 2, HALF)
    m = _mean_pool(xT, emb)
    return _mlp(m, W1, b1.reshape(1, HID), W2, b2.reshape(1, OUT))
